# baseline (speedup 1.0000x reference)
import functools

import jax
import jax.numpy as jnp
from jax import lax
from jax.experimental import pallas as pl
from jax.experimental.pallas import tpu as pltpu
from jax.experimental.pallas import tpu_sc as plsc

_SCALING = 1.0

_NC = 2
_NS = 16
_NW = _NC * _NS

_SB = 16384


def _tc_prep(V, D, R):
  pad = 128 - D - 2 * R
  n_blk = (V + _SB - 1) // _SB
  hb = _SB // 2

  def pack(wT_ref, a_ref):
    w = wT_ref[...].T
    a = a_ref[...].T
    lo = w.astype(jnp.bfloat16)
    hi = jnp.concatenate(
        [(a * _SCALING).astype(jnp.bfloat16),
         jnp.zeros((hb, D - R), jnp.bfloat16)], axis=1)
    lo_u = lax.bitcast_convert_type(lo, jnp.uint16).astype(jnp.uint32)
    hi_u = lax.bitcast_convert_type(hi, jnp.uint16).astype(jnp.uint32)
    return lax.bitcast_convert_type(lo_u | (hi_u << 16), jnp.int32)

  def body(wT1_ref, a1_ref, wT2_ref, a2_ref, t_ref):
    t_ref[...] = jnp.concatenate(
        [pack(wT1_ref, a1_ref), pack(wT2_ref, a2_ref)], axis=1)

  return pl.pallas_call(
      body,
      grid=(n_blk,),
      in_specs=[
          pl.BlockSpec((D, hb), lambda i: (0, 2 * i)),
          pl.BlockSpec((R, hb), lambda i: (0, 2 * i)),
          pl.BlockSpec((D, hb), lambda i: (0, jnp.minimum(2 * i + 1, V // hb))),
          pl.BlockSpec((R, hb), lambda i: (0, jnp.minimum(2 * i + 1, V // hb))),
      ],
      out_specs=pl.BlockSpec((hb, 128), lambda i: (i, 0)),
      out_shape=jax.ShapeDtypeStruct((n_blk * hb, 128), jnp.int32),
  ), n_blk * _SB


def _sc_gather(n_tok, Vp, chunk):
  tpw = n_tok // _NW
  n_chunks = tpw // chunk
  nsub = chunk // 128
  mesh = plsc.VectorSubcoreMesh(core_axis_name="c", subcore_axis_name="s")

  @functools.partial(
      pl.kernel,
      mesh=mesh,
      compiler_params=pltpu.CompilerParams(use_tc_tiling_on_sc=False),
      out_type=jax.ShapeDtypeStruct((n_tok, 64), jnp.int32),
      scratch_types=[
          pltpu.VMEM((nsub, 128), jnp.int32),
          pltpu.VMEM((nsub, 128), jnp.int32),
          pltpu.VMEM((chunk, 64), jnp.int32),
          pltpu.SemaphoreType.DMA,
      ],
  )
  def k(xf_hbm, t_hbm, g_hbm, idx_v, idx2_v, rows_v, sem):
    wid = lax.axis_index("s") * _NC + lax.axis_index("c")
    start = wid * tpw

    def body(ci, carry):
      off = start + ci * chunk
      pltpu.sync_copy(xf_hbm.at[pl.ds(off // 128, nsub)], idx_v)
      for j in range(nsub):
        for kk in range(8):
          sl = pl.ds(kk * 16, 16)
          v = idx_v[j, sl]
          idx2_v[j, sl] = (
              (v >> 14) << 14
          ) + ((v & 8191) << 1) + ((v >> 13) & 1)
      cps = [
          pltpu.async_copy(
              t_hbm.at[idx2_v.at[j]],
              rows_v.at[pl.ds(j * 128, 128)],
              sem,
          )
          for j in range(nsub)
      ]
      for cp in cps:
        cp.wait()
      pltpu.sync_copy(rows_v, g_hbm.at[pl.ds(off, chunk)])
      return carry

    lax.fori_loop(0, n_chunks, body, 0)

  return k


def _tc_combine(n_tok, L, D, bc):
  B = n_tok // L
  nb = B // bc
  hc = bc // 2

  def body(g_ref, mlo_ref, mhi_ref, out_ref):
    dims = (((1,), (1,)), ((), ()))
    for t in range(2):
      gu = lax.bitcast_convert_type(
          g_ref[pl.ds(t * hc, hc), :], jnp.uint32)
      lo = lax.bitcast_convert_type(
          (gu & 0xFFFF).astype(jnp.uint16), jnp.bfloat16)
      hi = lax.bitcast_convert_type(
          (gu >> 16).astype(jnp.uint16), jnp.bfloat16)
      out_ref[pl.ds(t * D, D), :hc] = lax.dot_general(
          mlo_ref[...], lo[:, :64], dims, preferred_element_type=jnp.float32
      ) + lax.dot_general(
          mhi_ref[...], hi[:, :64], dims, preferred_element_type=jnp.float32)
      out_ref[pl.ds(t * D, D), hc:] = lax.dot_general(
          mlo_ref[...], lo[:, 64:], dims, preferred_element_type=jnp.float32
      ) + lax.dot_general(
          mhi_ref[...], hi[:, 64:], dims, preferred_element_type=jnp.float32)

  return pl.pallas_call(
      body,
      grid=(L // 2,),
      in_specs=[
          pl.BlockSpec((2 * hc, 128), lambda i: (i, 0)),
          pl.BlockSpec((D, D), lambda i: (0, 0)),
          pl.BlockSpec((D, D), lambda i: (0, 0)),
      ],
      out_specs=pl.BlockSpec((2 * D, bc), lambda i: (i, 0)),
      out_shape=jax.ShapeDtypeStruct((L * D, B), jnp.float32),
  )


@jax.jit
def kernel(x, weight, lora_a, lora_b):
  B, L = x.shape
  V, D = weight.shape
  R = lora_a.shape[0]
  n_tok = B * L
  bc = 4096
  hc = bc // 2

  prep, Vp = _tc_prep(V, D, R)
  wT = weight.T
  table = prep(wT, lora_a, wT, lora_a)
  t64 = table.reshape(Vp, 64)

  xp = (x.T.astype(jnp.int32)
        .reshape(L, B // bc, 2, hc)
        .transpose(0, 1, 3, 2)
        .reshape(n_tok // 128, 128))
  g = _sc_gather(n_tok, Vp, chunk=1280)(xp, t64)
  g2 = g.reshape(n_tok // 2, 128)

  mlo = jnp.eye(D, dtype=jnp.bfloat16)
  mhi = jnp.concatenate(
      [lora_b, jnp.zeros((D, D - R), jnp.float32)], axis=1).astype(jnp.bfloat16)
  out2 = _tc_combine(n_tok, L, D, bc=bc)(g2, mlo, mhi)
  return out2.reshape(L, D, B).transpose(2, 0, 1)
